# baseline (speedup 1.0000x reference)
import jax
import jax.numpy as jnp
from jax import lax
from jax.experimental import pallas as pl
from jax.experimental.pallas import tpu as pltpu
from jax.experimental.pallas import tpu_sc as plsc

N_BATCH = 16
N_ATOMS = 512
N_NEIGH = 48
N_RADIUS = 16
N_CHANNEL = 16
CUTOFF = 6.0
N_Z = 10
L = 16
NW = 32
APW = N_BATCH * N_ATOMS // NW
RC = N_RADIUS * N_CHANNEL
NA = N_BATCH * N_ATOMS

_GDN = lax.GatherDimensionNumbers(
    offset_dims=(), collapsed_slice_dims=(0,), start_index_map=(0,))


def _bcast(vec, idx_vec):
    return lax.gather(vec, idx_vec[:, None], _GDN, (1,),
                      mode=lax.GatherScatterMode.PROMISE_IN_BOUNDS)


def _cos_poly(u):
    c = jnp.float32(-1.0 / 87178291200.0)
    c = c * u + jnp.float32(1.0 / 479001600.0)
    c = c * u + jnp.float32(-1.0 / 3628800.0)
    c = c * u + jnp.float32(1.0 / 40320.0)
    c = c * u + jnp.float32(-1.0 / 720.0)
    c = c * u + jnp.float32(1.0 / 24.0)
    c = c * u + jnp.float32(-0.5)
    return c * u + jnp.float32(1.0)


def _sc_body(coord_hbm, anum_hbm, nbr_hbm, negeta_hbm, rss_hbm,
             g_hbm, cxv, cyv, czv, anv, nbv, rssv, negv, gv):
    s = lax.axis_index("s")
    c = lax.axis_index("c")
    wid = s * 2 + c
    b = wid // 2
    h = wid % 2

    cbase = b * 3 * N_ATOMS
    pltpu.sync_copy(coord_hbm.at[pl.ds(cbase, N_ATOMS)], cxv)
    pltpu.sync_copy(coord_hbm.at[pl.ds(cbase + N_ATOMS, N_ATOMS)], cyv)
    pltpu.sync_copy(coord_hbm.at[pl.ds(cbase + 2 * N_ATOMS, N_ATOMS)], czv)
    pltpu.sync_copy(anum_hbm.at[pl.ds(b * N_ATOMS, N_ATOMS)], anv)
    nbase = (b * N_ATOMS + h * APW) * N_NEIGH
    pltpu.sync_copy(nbr_hbm.at[pl.ds(nbase, APW * N_NEIGH)], nbv)
    pltpu.sync_copy(rss_hbm, rssv)
    pltpu.sync_copy(negeta_hbm, negv)

    rss_vec = rssv[...]
    neg_vec = negv[...]
    iota16 = lax.iota(jnp.int32, L)
    iota_rz = iota16 * L
    ks = [jnp.full((L,), k, jnp.int32) for k in range(L)]
    zero16 = jnp.zeros((L,), jnp.float32)
    half = jnp.float32(0.5)
    three_half = jnp.float32(1.5)
    magic = jnp.int32(0x5F3759DF)

    def atom(i):
        nb0 = i * N_NEIGH
        gb = i * RC
        gbase = iota_rz + gb
        for r in range(N_RADIUS):
            gv[pl.ds(gb + r * L, L)] = zero16
        n_i = h * APW + i
        own = jnp.full((L,), n_i, jnp.int32)
        xi = plsc.load_gather(cxv, [own])
        yi = plsc.load_gather(cyv, [own])
        zi = plsc.load_gather(czv, [own])
        for g in range(N_NEIGH // L):
            nbr = nbv[pl.ds(nb0 + g * L, L)]
            zng = plsc.load_gather(anv, [nbr])
            xj = plsc.load_gather(cxv, [nbr])
            yj = plsc.load_gather(cyv, [nbr])
            zj = plsc.load_gather(czv, [nbr])
            dx = xj - xi
            dy = yj - yi
            dz = zj - zi
            d2 = (dx * dx + dy * dy) + (dz * dz + jnp.float32(1e-12))
            y = plsc.bitcast(magic - (plsc.bitcast(d2, jnp.int32) >> 1),
                             jnp.float32)
            hd2 = half * d2
            y = y * (three_half - hd2 * y * y)
            y = y * (three_half - hd2 * y * y)
            y = y * (three_half - hd2 * y * y)
            dd = d2 * y
            inb = d2 < jnp.float32(CUTOFF * CUTOFF)
            dcl = jnp.minimum(dd, jnp.float32(CUTOFF))
            x = dcl * jnp.float32(3.141592653589793 / CUTOFF)
            cosv = _cos_poly(x * x)
            cut = jnp.where(inb, half * (cosv + jnp.float32(1.0)), zero16)
            for k in range(L):
                db = _bcast(dcl, ks[k])
                cb = _bcast(cut, ks[k])
                zb = _bcast(zng, ks[k])
                tt = db - rss_vec
                fk = jnp.exp(tt * tt * neg_vec) * cb
                plsc.addupdate_scatter(gv, [gbase + zb], fk)

    def body(i, carry):
        i0 = i * 2
        atom(i0)
        atom(i0 + 1)
        return carry

    lax.fori_loop(0, APW // 2, body, 0)
    obase = (b * N_ATOMS + h * APW) * RC
    pltpu.sync_copy(gv, g_hbm.at[pl.ds(obase, APW * RC)])


def _tc_body(g_ref, bd_ref, o_ref):
    o_ref[...] = jnp.dot(g_ref[...], bd_ref[...],
                         preferred_element_type=jnp.float32,
                         precision=lax.Precision.HIGHEST)


@jax.jit
def _run(coord_t, anum, nbr_flat, bd, negeta, rss):
    mesh = plsc.VectorSubcoreMesh(core_axis_name="c", subcore_axis_name="s")
    sc = pl.kernel(
        _sc_body,
        out_type=jax.ShapeDtypeStruct((NA * RC,), jnp.float32),
        mesh=mesh,
        compiler_params=pltpu.CompilerParams(needs_layout_passes=False),
        scratch_types=[
            pltpu.VMEM((N_ATOMS,), jnp.float32),
            pltpu.VMEM((N_ATOMS,), jnp.float32),
            pltpu.VMEM((N_ATOMS,), jnp.float32),
            pltpu.VMEM((N_ATOMS,), jnp.int32),
            pltpu.VMEM((APW * N_NEIGH,), jnp.int32),
            pltpu.VMEM((L,), jnp.float32),
            pltpu.VMEM((L,), jnp.float32),
            pltpu.VMEM((APW * RC,), jnp.float32),
        ],
    )
    g_all = sc(coord_t, anum, nbr_flat, negeta, rss)
    g2 = g_all.reshape(NA, RC)
    blk = 1024
    out = pl.pallas_call(
        _tc_body,
        out_shape=jax.ShapeDtypeStruct((NA, RC), jnp.float32),
        grid=(NA // blk,),
        in_specs=[
            pl.BlockSpec((blk, RC), lambda i: (i, 0)),
            pl.BlockSpec((RC, RC), lambda i: (0, 0)),
        ],
        out_specs=pl.BlockSpec((blk, RC), lambda i: (i, 0)),
    )(g2, bd)
    return out


def kernel(coordinate, atomic_number, neighbor, emb_table, etas, rss):
    coord_t = coordinate.astype(jnp.float32).transpose(0, 2, 1).reshape(-1)
    anum = atomic_number.astype(jnp.int32).reshape(-1)
    nbr_flat = neighbor.astype(jnp.int32).reshape(-1)
    emb_pad = jnp.zeros((L, N_CHANNEL), jnp.float32)
    emb_pad = emb_pad.at[:N_Z].set(emb_table.astype(jnp.float32))
    bd = jnp.kron(jnp.eye(L, dtype=jnp.float32), emb_pad)
    out = _run(coord_t, anum, nbr_flat, bd,
               (-etas).astype(jnp.float32), rss.astype(jnp.float32))
    return out.reshape(N_BATCH, N_ATOMS, RC)


# speedup vs baseline: 1.0656x; 1.0656x over previous
import jax
import jax.numpy as jnp
from jax import lax
from jax.experimental import pallas as pl
from jax.experimental.pallas import tpu as pltpu
from jax.experimental.pallas import tpu_sc as plsc

N_BATCH = 16
N_ATOMS = 512
N_NEIGH = 48
N_RADIUS = 16
N_CHANNEL = 16
CUTOFF = 6.0
N_Z = 10
L = 16
NW = 32
APW = N_BATCH * N_ATOMS // NW
RC = N_RADIUS * N_CHANNEL
NA = N_BATCH * N_ATOMS

_GDN = lax.GatherDimensionNumbers(
    offset_dims=(), collapsed_slice_dims=(0,), start_index_map=(0,))


def _bcast(vec, idx_vec):
    return lax.gather(vec, idx_vec[:, None], _GDN, (1,),
                      mode=lax.GatherScatterMode.PROMISE_IN_BOUNDS)


def _cos_poly(u):
    c = jnp.float32(-1.0 / 87178291200.0)
    c = c * u + jnp.float32(1.0 / 479001600.0)
    c = c * u + jnp.float32(-1.0 / 3628800.0)
    c = c * u + jnp.float32(1.0 / 40320.0)
    c = c * u + jnp.float32(-1.0 / 720.0)
    c = c * u + jnp.float32(1.0 / 24.0)
    c = c * u + jnp.float32(-0.5)
    return c * u + jnp.float32(1.0)


def _sc_body(coord_hbm, anum_hbm, nbr_hbm, negeta_hbm, rss_hbm,
             g_hbm, cxv, cyv, czv, anv, nbv, rssv, negv, gv):
    s = lax.axis_index("s")
    c = lax.axis_index("c")
    wid = s * 2 + c
    b = wid // 2
    h = wid % 2

    cbase = b * 3 * N_ATOMS
    pltpu.sync_copy(coord_hbm.at[pl.ds(cbase, N_ATOMS)], cxv)
    pltpu.sync_copy(coord_hbm.at[pl.ds(cbase + N_ATOMS, N_ATOMS)], cyv)
    pltpu.sync_copy(coord_hbm.at[pl.ds(cbase + 2 * N_ATOMS, N_ATOMS)], czv)
    pltpu.sync_copy(anum_hbm.at[pl.ds(b * N_ATOMS, N_ATOMS)], anv)
    nbase = (b * N_ATOMS + h * APW) * N_NEIGH
    pltpu.sync_copy(nbr_hbm.at[pl.ds(nbase, APW * N_NEIGH)], nbv)
    pltpu.sync_copy(rss_hbm, rssv)
    pltpu.sync_copy(negeta_hbm, negv)

    rss_vec = rssv[...]
    neg_vec = negv[...]
    iota16 = lax.iota(jnp.int32, L)
    iota_rz = iota16 * L
    ks = [jnp.full((L,), k, jnp.int32) for k in range(L)]
    zero16 = jnp.zeros((L,), jnp.float32)
    half = jnp.float32(0.5)
    three_half = jnp.float32(1.5)
    magic = jnp.int32(0x5F3759DF)

    def atom(i):
        nb0 = i * N_NEIGH
        gb = i * RC
        gbase = iota_rz + gb
        for r in range(N_RADIUS):
            gv[pl.ds(gb + r * L, L)] = zero16
        n_i = h * APW + i
        own = jnp.full((L,), n_i, jnp.int32)
        xi = plsc.load_gather(cxv, [own])
        yi = plsc.load_gather(cyv, [own])
        zi = plsc.load_gather(czv, [own])
        for g in range(N_NEIGH // L):
            nbr = nbv[pl.ds(nb0 + g * L, L)]
            zng = plsc.load_gather(anv, [nbr])
            xj = plsc.load_gather(cxv, [nbr])
            yj = plsc.load_gather(cyv, [nbr])
            zj = plsc.load_gather(czv, [nbr])
            dx = xj - xi
            dy = yj - yi
            dz = zj - zi
            d2 = (dx * dx + dy * dy) + (dz * dz + jnp.float32(1e-12))
            y = plsc.bitcast(magic - (plsc.bitcast(d2, jnp.int32) >> 1),
                             jnp.float32)
            hd2 = half * d2
            y = y * (three_half - hd2 * y * y)
            y = y * (three_half - hd2 * y * y)
            y = y * (three_half - hd2 * y * y)
            dd = d2 * y
            inb = d2 < jnp.float32(CUTOFF * CUTOFF)
            dcl = jnp.minimum(dd, jnp.float32(CUTOFF))
            x = dcl * jnp.float32(3.141592653589793 / CUTOFF)
            cosv = _cos_poly(x * x)
            cut = jnp.where(inb, half * (cosv + jnp.float32(1.0)), zero16)
            for k in range(L):
                db = _bcast(dcl, ks[k])
                cb = _bcast(cut, ks[k])
                zb = _bcast(zng, ks[k])
                tt = db - rss_vec
                fk = jnp.exp(tt * tt * neg_vec) * cb
                plsc.addupdate_scatter(gv, [gbase + zb], fk)

    @plsc.parallel_loop(0, APW, unroll=2)
    def _loop(i):
        atom(i)
    obase = (b * N_ATOMS + h * APW) * RC
    pltpu.sync_copy(gv, g_hbm.at[pl.ds(obase, APW * RC)])


def _tc_body(g_ref, bd_ref, o_ref):
    o_ref[...] = jnp.dot(g_ref[...], bd_ref[...],
                         preferred_element_type=jnp.float32)


@jax.jit
def _run(coord_t, anum, nbr_flat, bd, negeta, rss):
    mesh = plsc.VectorSubcoreMesh(core_axis_name="c", subcore_axis_name="s")
    sc = pl.kernel(
        _sc_body,
        out_type=jax.ShapeDtypeStruct((NA * RC,), jnp.float32),
        mesh=mesh,
        compiler_params=pltpu.CompilerParams(needs_layout_passes=False),
        scratch_types=[
            pltpu.VMEM((N_ATOMS,), jnp.float32),
            pltpu.VMEM((N_ATOMS,), jnp.float32),
            pltpu.VMEM((N_ATOMS,), jnp.float32),
            pltpu.VMEM((N_ATOMS,), jnp.int32),
            pltpu.VMEM((APW * N_NEIGH,), jnp.int32),
            pltpu.VMEM((L,), jnp.float32),
            pltpu.VMEM((L,), jnp.float32),
            pltpu.VMEM((APW * RC,), jnp.float32),
        ],
    )
    g_all = sc(coord_t, anum, nbr_flat, negeta, rss)
    g2 = g_all.reshape(NA, RC)
    blk = 1024
    out = pl.pallas_call(
        _tc_body,
        out_shape=jax.ShapeDtypeStruct((NA, RC), jnp.float32),
        grid=(NA // blk,),
        in_specs=[
            pl.BlockSpec((blk, RC), lambda i: (i, 0)),
            pl.BlockSpec((RC, RC), lambda i: (0, 0)),
        ],
        out_specs=pl.BlockSpec((blk, RC), lambda i: (i, 0)),
    )(g2, bd)
    return out


def kernel(coordinate, atomic_number, neighbor, emb_table, etas, rss):
    coord_t = coordinate.astype(jnp.float32).transpose(0, 2, 1).reshape(-1)
    anum = atomic_number.astype(jnp.int32).reshape(-1)
    nbr_flat = neighbor.astype(jnp.int32).reshape(-1)
    emb_pad = jnp.zeros((L, N_CHANNEL), jnp.float32)
    emb_pad = emb_pad.at[:N_Z].set(emb_table.astype(jnp.float32))
    bd = jnp.kron(jnp.eye(L, dtype=jnp.float32), emb_pad)
    out = _run(coord_t, anum, nbr_flat, bd,
               (-etas).astype(jnp.float32), rss.astype(jnp.float32))
    return out.reshape(N_BATCH, N_ATOMS, RC)
